# Initial kernel scaffold; baseline (speedup 1.0000x reference)
import jax
import jax.numpy as jnp
from jax.experimental import pallas as pl


def kernel(x, edge_index, W1, a_src1, a_dst1, b1, W2, a_src2, a_dst2, b2):
    raise NotImplementedError("write your pallas kernel here")


# speedup vs baseline: 1.1689x; 1.1689x over previous
import jax
import jax.numpy as jnp
from jax.experimental import pallas as pl
from jax.experimental.pallas import tpu as pltpu

N = 10000
DIM = 128
HID = 8
HEADS = 8
NCLS = 2


def _matmul_kernel(x_ref, w_ref, o_ref):
    o_ref[...] = jnp.dot(x_ref[...], w_ref[...],
                         preferred_element_type=jnp.float32)


def _matmul(x, w):
    n, k = x.shape
    m = w.shape[1]
    mp = max(128, m)
    if mp != m:
        w = jnp.pad(w, ((0, 0), (0, mp - m)))
    blk = 1000
    out = _matmul_call(x, w, n, k, mp, blk)
    return out[:, :m] if mp != m else out


def _matmul_call(x, w, n, k, m, blk):
    return pl.pallas_call(
        _matmul_kernel,
        grid=(n // blk,),
        in_specs=[
            pl.BlockSpec((blk, k), lambda i: (i, 0)),
            pl.BlockSpec((k, m), lambda i: (0, 0)),
        ],
        out_specs=pl.BlockSpec((blk, m), lambda i: (i, 0)),
        out_shape=jax.ShapeDtypeStruct((n, m), jnp.float32),
    )(x, w)


def _gat_conv(x, src, dst, W, a_src, a_dst, b, heads, out_ch):
    n = x.shape[0]
    h = _matmul(x, W).reshape(n, heads, out_ch)
    alpha_src = jnp.sum(h * a_src[None, :, :], axis=-1)
    alpha_dst = jnp.sum(h * a_dst[None, :, :], axis=-1)
    e = alpha_src[src] + alpha_dst[dst]
    e = jax.nn.leaky_relu(e, negative_slope=0.2)
    ex = jnp.exp(e)
    s = jax.ops.segment_sum(ex, dst, num_segments=n)
    num = jax.ops.segment_sum(h[src] * ex[:, :, None], dst, num_segments=n)
    out = num / (s[:, :, None] + 1e-16)
    return out.reshape(n, heads * out_ch) + b


def kernel(x, edge_index, W1, a_src1, a_dst1, b1, W2, a_src2, a_dst2, b2):
    loop = jnp.arange(N, dtype=edge_index.dtype)
    src = jnp.concatenate([edge_index[0], loop])
    dst = jnp.concatenate([edge_index[1], loop])
    h = _gat_conv(x, src, dst, W1, a_src1, a_dst1, b1, HEADS, HID)
    h = jax.nn.elu(h)
    out = _gat_conv(h, src, dst, W2, a_src2, a_dst2, b2, 1, NCLS)
    return out


# speedup vs baseline: 81.8117x; 69.9891x over previous
import functools

import jax
import jax.numpy as jnp
from jax import lax
from jax.experimental import pallas as pl
from jax.experimental.pallas import tpu as pltpu
from jax.experimental.pallas import tpu_sc as plsc

N = 10000
E = 320000
DIM = 128
HID = 8
HEADS = 8
NCLS = 2

NP = 10240
ETOT = E + N
CH = 81
EP = 32 * CH * 128
BLK = NP // 16
W1ROW = 80
W2ROW = 16

_mesh = plsc.VectorSubcoreMesh(core_axis_name="c", subcore_axis_name="s")


def _gath16(v, idx):
    dn = lax.GatherDimensionNumbers(
        offset_dims=(), collapsed_slice_dims=(0,), start_index_map=(0,))
    return lax.gather(v, idx[:, None], dn, (1,),
                      mode=lax.GatherScatterMode.PROMISE_IN_BOUNDS)


def _prep_body(x_ref, ms_ref, md_ref, s_ref, d_ref):
    xb = x_ref[...]
    s_ref[...] = jnp.dot(xb, ms_ref[...], preferred_element_type=jnp.float32)
    d_ref[...] = jnp.dot(xb, md_ref[...], preferred_element_type=jnp.float32)


def _prep(xp, ms, md):
    k = xp.shape[1]
    ws, wd = ms.shape[1], md.shape[1]
    return pl.pallas_call(
        _prep_body,
        grid=(16,),
        in_specs=[
            pl.BlockSpec((BLK, k), lambda i: (i, 0)),
            pl.BlockSpec((k, ws), lambda i: (0, 0)),
            pl.BlockSpec((k, wd), lambda i: (0, 0)),
        ],
        out_specs=[
            pl.BlockSpec((BLK, ws), lambda i: (i, 0)),
            pl.BlockSpec((BLK, wd), lambda i: (i, 0)),
        ],
        out_shape=[
            jax.ShapeDtypeStruct((NP, ws), jnp.float32),
            jax.ShapeDtypeStruct((NP, wd), jnp.float32),
        ],
    )(xp, ms, md)


def _mid_body(acc_ref, p1_ref, p2_ref, r8_ref, m1_ref, m2_ref, b1_ref,
              s_ref, d_ref):
    num = acc_ref[0] + acc_ref[1]
    hb = jnp.dot(num, p1_ref[...], preferred_element_type=jnp.float32)
    s8 = jnp.dot(num, p2_ref[...], preferred_element_type=jnp.float32)
    s64 = jnp.dot(s8, r8_ref[...], preferred_element_type=jnp.float32)
    g = hb / (s64 + 1e-16) + b1_ref[...]
    el = jnp.where(g > 0.0, g, jnp.exp(g) - 1.0)
    s_ref[...] = jnp.dot(el, m1_ref[...], preferred_element_type=jnp.float32)
    d_ref[...] = jnp.dot(el, m2_ref[...], preferred_element_type=jnp.float32)


def _mid(acc1, p1, p2, r8, m1, m2, b1r):
    return pl.pallas_call(
        _mid_body,
        grid=(16,),
        in_specs=[
            pl.BlockSpec((2, BLK, W1ROW), lambda i: (0, i, 0)),
            pl.BlockSpec((W1ROW, 64), lambda i: (0, 0)),
            pl.BlockSpec((W1ROW, 8), lambda i: (0, 0)),
            pl.BlockSpec((8, 64), lambda i: (0, 0)),
            pl.BlockSpec((64, W2ROW), lambda i: (0, 0)),
            pl.BlockSpec((64, W2ROW), lambda i: (0, 0)),
            pl.BlockSpec((1, 64), lambda i: (0, 0)),
        ],
        out_specs=[
            pl.BlockSpec((BLK, W2ROW), lambda i: (i, 0)),
            pl.BlockSpec((BLK, W2ROW), lambda i: (i, 0)),
        ],
        out_shape=[
            jax.ShapeDtypeStruct((NP, W2ROW), jnp.float32),
            jax.ShapeDtypeStruct((NP, W2ROW), jnp.float32),
        ],
    )(acc1, p1, p2, r8, m1, m2, b1r)


def _fin_body(acc_ref, e01_ref, e2_ref, b2_ref, o_ref):
    num = acc_ref[0] + acc_ref[1]
    nk = jnp.dot(num, e01_ref[...], preferred_element_type=jnp.float32)
    sv = jnp.dot(num, e2_ref[...], preferred_element_type=jnp.float32)
    o_ref[...] = nk / (sv + 1e-16) + b2_ref[...]


def _fin(acc2, e01, e2, b2p):
    return pl.pallas_call(
        _fin_body,
        grid=(16,),
        in_specs=[
            pl.BlockSpec((2, BLK, W2ROW), lambda i: (0, i, 0)),
            pl.BlockSpec((W2ROW, 128), lambda i: (0, 0)),
            pl.BlockSpec((W2ROW, 128), lambda i: (0, 0)),
            pl.BlockSpec((1, 128), lambda i: (0, 0)),
        ],
        out_specs=pl.BlockSpec((BLK, 128), lambda i: (i, 0)),
        out_shape=jax.ShapeDtypeStruct((NP, 128), jnp.float32),
    )(acc2, e01, e2, b2p)


def _edge_call(stab, dtab, sidx, didx, width, edge_fn):
    @functools.partial(
        pl.kernel,
        out_type=jax.ShapeDtypeStruct((2, NP, width), jnp.float32),
        mesh=_mesh,
        compiler_params=pltpu.CompilerParams(use_tc_tiling_on_sc=False),
        scratch_types=[
            pltpu.VMEM((CH, 128), jnp.int32),
            pltpu.VMEM((CH, 128), jnp.int32),
            pltpu.VMEM((128, width), jnp.float32),
            pltpu.VMEM((128, 16), jnp.float32),
            pltpu.VMEM((128, width), jnp.float32),
            pltpu.VMEM_SHARED((NP, width), jnp.float32),
            pltpu.SemaphoreType.DMA,
            pltpu.SemaphoreType.DMA,
        ],
    )
    def k(stab_hbm, dtab_hbm, sidx_hbm, didx_hbm, out_hbm,
          idx_s, idx_d, rows, drows, outb, acc, sem1, sem2):
        cid = lax.axis_index("c")
        sid = lax.axis_index("s")
        wid = cid * 16 + sid
        nvec = width // 16
        zeros16 = jnp.zeros((16,), jnp.float32)

        def zrow(i, _):
            for j in range(nvec):
                outb[i, pl.ds(16 * j, 16)] = zeros16
            return 0
        lax.fori_loop(0, 128, zrow, 0)
        base = sid * BLK
        for kc in range(BLK // 128):
            pltpu.sync_copy(outb, acc.at[pl.ds(base + kc * 128, 128)])
        plsc.subcore_barrier()

        pltpu.sync_copy(sidx_hbm.at[wid], idx_s)
        pltpu.sync_copy(didx_hbm.at[wid], idx_d)

        def chunk(ci, _):
            cp1 = pltpu.async_copy(stab_hbm.at[idx_s.at[ci]], rows, sem1)
            cp2 = pltpu.async_copy(dtab_hbm.at[idx_d.at[ci]], drows, sem2)
            cp1.wait()
            cp2.wait()
            lax.fori_loop(0, 128, edge_fn(rows, drows, outb), 0)
            pltpu.sync_copy(outb, acc.at[idx_d.at[ci]], add=True)
            return 0
        lax.fori_loop(0, CH, chunk, 0)
        plsc.subcore_barrier()

        pltpu.sync_copy(acc.at[pl.ds(base, BLK)],
                        out_hbm.at[cid, pl.ds(base, BLK)])

    return k(stab, dtab, sidx, didx)


def _edge1_fn(rows, drows, outb):
    iota = lax.iota(jnp.int32, 16)

    def edge(i, _):
        asv = rows[i, pl.ds(64, 16)]
        adv = drows[i, pl.ds(0, 16)]
        e = asv + adv
        e = jnp.where(e >= 0.0, e, e * 0.2)
        w = jnp.exp(e)
        outb[i, pl.ds(64, 16)] = w
        for j in range(4):
            wb = _gath16(w, jnp.where(iota < 8, 2 * j, 2 * j + 1))
            outb[i, pl.ds(16 * j, 16)] = rows[i, pl.ds(16 * j, 16)] * wb
        return 0
    return edge


def _edge2_fn(rows, drows, outb):
    iota = lax.iota(jnp.int32, 16)

    def edge(i, _):
        sv = rows[i, pl.ds(0, 16)]
        dv = drows[i, pl.ds(0, 16)]
        e = _gath16(sv, iota * 0 + 2) + _gath16(dv, iota * 0)
        e = jnp.where(e >= 0.0, e, e * 0.2)
        w = jnp.exp(e)
        sel = jnp.where(iota == 2, 1.0, jnp.where(iota < 2, sv, 0.0))
        outb[i, pl.ds(0, 16)] = w * sel
        return 0
    return edge


def kernel(x, edge_index, W1, a_src1, a_dst1, b1, W2, a_src2, a_dst2, b2):
    f32 = jnp.float32
    loop = jnp.arange(N, dtype=jnp.int32)
    src = jnp.concatenate([edge_index[0].astype(jnp.int32), loop])
    dst = jnp.concatenate([edge_index[1].astype(jnp.int32), loop])
    pad = jnp.full((EP - ETOT,), N, jnp.int32)
    src3 = jnp.concatenate([src, pad]).reshape(32, CH, 128)
    dst3 = jnp.concatenate([dst, pad]).reshape(32, CH, 128)

    ar = jnp.arange(64)
    a1 = jnp.zeros((64, HEADS), f32).at[ar, ar // HID].set(a_src1.reshape(-1))
    a1d = jnp.zeros((64, HEADS), f32).at[ar, ar // HID].set(a_dst1.reshape(-1))
    ms1 = jnp.concatenate([W1, W1 @ a1, jnp.zeros((DIM, 8), f32)], axis=1)
    md1 = jnp.concatenate([W1 @ a1d, jnp.zeros((DIM, 8), f32)], axis=1)
    p1 = jnp.zeros((W1ROW, 64), f32).at[ar, ar].set(1.0)
    p2 = jnp.zeros((W1ROW, 8), f32).at[64 + jnp.arange(8), jnp.arange(8)].set(1.0)
    r8 = jnp.zeros((8, 64), f32).at[ar // HID, ar].set(1.0)
    m1 = jnp.concatenate([W2, W2 @ a_src2.T, jnp.zeros((64, 13), f32)], axis=1)
    m2 = jnp.concatenate([W2 @ a_dst2.T, jnp.zeros((64, 15), f32)], axis=1)
    e01 = jnp.zeros((W2ROW, 128), f32).at[jnp.arange(2), jnp.arange(2)].set(1.0)
    e2 = jnp.zeros((W2ROW, 128), f32).at[2, :].set(1.0)
    b2p = jnp.zeros((1, 128), f32).at[0, :2].set(b2)

    xp = jnp.zeros((NP, DIM), f32).at[:N].set(x)
    stab1, dtab1 = _prep(xp, ms1, md1)
    acc1 = _edge_call(stab1, dtab1, src3, dst3, W1ROW, _edge1_fn)
    stab2, dtab2 = _mid(acc1, p1, p2, r8, m1, m2, b1.reshape(1, 64))
    acc2 = _edge_call(stab2, dtab2, src3, dst3, W2ROW, _edge2_fn)
    outp = _fin(acc2, e01, e2, b2p)
    return outp[:N, :NCLS]
